# Initial kernel scaffold; baseline (speedup 1.0000x reference)
import jax
import jax.numpy as jnp
from jax.experimental import pallas as pl


def kernel(x, edge_index, edge_attr, eps1, We1, be1, W1a, b1a, g1a, bt1a, W1b, b1b, gbn1, bbn1, eps2, We2, be2, W2a, b2a, g2a, bt2a, W2b, b2b, gbn2, bbn2, W3, b3):
    raise NotImplementedError("write your pallas kernel here")


# speedup vs baseline: 2.8125x; 2.8125x over previous
import functools

import jax
import jax.numpy as jnp
from jax import lax
from jax.experimental import pallas as pl
from jax.experimental.pallas import tpu as pltpu
from jax.experimental.pallas import tpu_sc as plsc

N = 10000
E = 320000
D_NODE = 128
D_EDGE = 16

NC = 2
NS = 16
NW = NC * NS
EPW = E // NW
C = 80
G = EPW // C
NP = 10240
RPT = NP // NS
ZR = 128


def _make_edge_kernel(D, stage_x):
    mesh = plsc.VectorSubcoreMesh(core_axis_name="c", subcore_axis_name="s",
                                  num_cores=NC, num_subcores=NS)

    scratch = [
        pltpu.VMEM((C,), jnp.int32),
        pltpu.VMEM((C,), jnp.int32),
        pltpu.VMEM((C, D), jnp.float32),
        pltpu.VMEM((C, D), jnp.float32),
        pltpu.VMEM((ZR, D), jnp.float32),
        pltpu.VMEM_SHARED((NP, D), jnp.float32),
        pltpu.SemaphoreType.DMA,
        pltpu.SemaphoreType.DMA,
    ]
    if stage_x:
        scratch.append(pltpu.VMEM_SHARED((NP, D), jnp.float32))

    @functools.partial(
        pl.kernel,
        out_type=jax.ShapeDtypeStruct((NC * NP, D), jnp.float32),
        mesh=mesh,
        scratch_types=scratch,
    )
    def edge_kernel(x_hbm, src_hbm, dst_hbm, elin_hbm, out_hbm,
                    src_v, dst_v, rows_v, elin_v, zero_v, aggr_sh,
                    sem1, sem2, *maybe_x_sh):
        c = lax.axis_index("c")
        s = lax.axis_index("s")
        nv = D // 16

        row0 = s * RPT
        if stage_x:
            x_sh = maybe_x_sh[0]
            for k in range(RPT // ZR):
                sl = pl.ds(row0 + k * ZR, ZR)
                pltpu.sync_copy(x_hbm.at[sl], zero_v)
                pltpu.sync_copy(zero_v, x_sh.at[sl])
            gather_src = x_sh
        else:
            gather_src = x_hbm

        def zbody(i, carry):
            for j in range(nv):
                zero_v[i, pl.ds(j * 16, 16)] = jnp.zeros((16,), jnp.float32)
            return carry

        lax.fori_loop(0, ZR, zbody, 0)
        for k in range(RPT // ZR):
            pltpu.sync_copy(zero_v, aggr_sh.at[pl.ds(row0 + k * ZR, ZR)])
        plsc.subcore_barrier()

        wid = c * NS + s
        e0 = wid * EPW

        def gbody(g, carry):
            base = e0 + g * C
            pltpu.sync_copy(src_hbm.at[pl.ds(base, C)], src_v)
            pltpu.sync_copy(dst_hbm.at[pl.ds(base, C)], dst_v)
            cp1 = pltpu.async_copy(gather_src.at[src_v], rows_v, sem1)
            cp2 = pltpu.async_copy(elin_hbm.at[pl.ds(base, C)], elin_v, sem2)
            cp1.wait()
            cp2.wait()

            def rbody(r, carry2):
                for j in range(nv):
                    sl = pl.ds(j * 16, 16)
                    rows_v[r, sl] = jnp.maximum(
                        rows_v[r, sl] + elin_v[r, sl], 0.0)
                return carry2

            lax.fori_loop(0, C, rbody, 0)
            pltpu.sync_copy(rows_v, aggr_sh.at[dst_v], add=True)
            return carry

        lax.fori_loop(0, G, gbody, 0)
        plsc.subcore_barrier()

        for k in range(RPT // ZR):
            pltpu.sync_copy(aggr_sh.at[pl.ds(row0 + k * ZR, ZR)], zero_v)
            pltpu.sync_copy(zero_v,
                            out_hbm.at[pl.ds(c * NP + row0 + k * ZR, ZR)])

    return edge_kernel


_edge_kernel_128 = _make_edge_kernel(D_NODE, stage_x=False)


_BE = 6400


def _elin_body(ea_ref, We1_ref, be1_ref, We2_ref, be2_ref, o1_ref, o2_ref):
    ea = ea_ref[...]
    o1_ref[...] = (jnp.dot(ea, We1_ref[...],
                           preferred_element_type=jnp.float32)
                   + be1_ref[...])
    o2_ref[...] = (jnp.dot(ea, We2_ref[...],
                           preferred_element_type=jnp.float32)
                   + be2_ref[...])


def _elin(ea, We1, be1, We2, be2):
    grid = (E // _BE,)
    return pl.pallas_call(
        _elin_body,
        grid=grid,
        in_specs=[
            pl.BlockSpec((_BE, D_EDGE), lambda i: (i, 0)),
            pl.BlockSpec((D_EDGE, D_NODE), lambda i: (0, 0)),
            pl.BlockSpec((1, D_NODE), lambda i: (0, 0)),
            pl.BlockSpec((D_EDGE, D_NODE), lambda i: (0, 0)),
            pl.BlockSpec((1, D_NODE), lambda i: (0, 0)),
        ],
        out_specs=[
            pl.BlockSpec((_BE, D_NODE), lambda i: (i, 0)),
            pl.BlockSpec((_BE, D_NODE), lambda i: (i, 0)),
        ],
        out_shape=[
            jax.ShapeDtypeStruct((E, D_NODE), jnp.float32),
            jax.ShapeDtypeStruct((E, D_NODE), jnp.float32),
        ],
    )(ea, We1, be1.reshape(1, -1),
      jnp.pad(We2, ((0, 0), (0, D_NODE - 32))),
      jnp.pad(be2, (0, D_NODE - 32)).reshape(1, -1))


def _bn_gelu(t, g, b):
    mu = jnp.mean(t, axis=0, keepdims=True)
    var = jnp.mean(jnp.square(t), axis=0, keepdims=True) - jnp.square(mu)
    t = (t - mu) * lax.rsqrt(var + 1e-5) * g + b
    return 0.5 * t * (1.0 + lax.erf(t * 0.7071067811865476))


def _node1_body(x_ref, p_ref, eps_ref, W1a_ref, b1a_ref, g1a_ref, bt1a_ref,
                W1b_ref, b1b_ref, gbn1_ref, bbn1_ref, o_ref):
    x = x_ref[...]
    z = (1.0 + eps_ref[0]) * x + p_ref[0:N] + p_ref[NP:NP + N]
    t = jnp.dot(z, W1a_ref[...], preferred_element_type=jnp.float32)
    t = t + b1a_ref[...]
    t = _bn_gelu(t, g1a_ref[...], bt1a_ref[...])
    t = jnp.dot(t, W1b_ref[...], preferred_element_type=jnp.float32)
    t = t + b1b_ref[...]
    r = _bn_gelu(t, gbn1_ref[...], bbn1_ref[...])
    r = jnp.concatenate([r, jnp.zeros((N, D_NODE - 32), jnp.float32)], axis=1)
    o_ref[...] = jnp.concatenate(
        [r, jnp.zeros((NP - N, D_NODE), jnp.float32)], axis=0)


def _node1(x, p, eps1, W1a, b1a, g1a, bt1a, W1b, b1b, gbn1, bbn1):
    return pl.pallas_call(
        _node1_body,
        in_specs=[
            pl.BlockSpec(memory_space=pltpu.VMEM),
            pl.BlockSpec(memory_space=pltpu.VMEM),
            pl.BlockSpec(memory_space=pltpu.SMEM),
            pl.BlockSpec(memory_space=pltpu.VMEM),
            pl.BlockSpec(memory_space=pltpu.VMEM),
            pl.BlockSpec(memory_space=pltpu.VMEM),
            pl.BlockSpec(memory_space=pltpu.VMEM),
            pl.BlockSpec(memory_space=pltpu.VMEM),
            pl.BlockSpec(memory_space=pltpu.VMEM),
            pl.BlockSpec(memory_space=pltpu.VMEM),
            pl.BlockSpec(memory_space=pltpu.VMEM),
        ],
        out_shape=jax.ShapeDtypeStruct((NP, D_NODE), jnp.float32),
    )(x, p, eps1.reshape(1), W1a, b1a.reshape(1, -1), g1a.reshape(1, -1),
      bt1a.reshape(1, -1), W1b, b1b.reshape(1, -1), gbn1.reshape(1, -1),
      bbn1.reshape(1, -1))


def _node2_body(h_ref, p_ref, eps_ref, W2a_ref, b2a_ref, g2a_ref, bt2a_ref,
                W2b_ref, b2b_ref, gbn2_ref, bbn2_ref, W3_ref, b3_ref, o_ref):
    h = h_ref[0:N, 0:32]
    z = (1.0 + eps_ref[0]) * h + p_ref[0:N, 0:32] + p_ref[NP:NP + N, 0:32]
    t = jnp.dot(z, W2a_ref[...], preferred_element_type=jnp.float32)
    t = t + b2a_ref[...]
    t = _bn_gelu(t, g2a_ref[...], bt2a_ref[...])
    t = jnp.dot(t, W2b_ref[...], preferred_element_type=jnp.float32)
    t = t + b2b_ref[...]
    t = _bn_gelu(t, gbn2_ref[...], bbn2_ref[...])
    o_ref[...] = (jnp.dot(t, W3_ref[...],
                          preferred_element_type=jnp.float32)
                  + b3_ref[...])


def _node2(h, p, eps2, W2a, b2a, g2a, bt2a, W2b, b2b, gbn2, bbn2, W3, b3):
    return pl.pallas_call(
        _node2_body,
        in_specs=[pl.BlockSpec(memory_space=pltpu.VMEM),
                  pl.BlockSpec(memory_space=pltpu.VMEM),
                  pl.BlockSpec(memory_space=pltpu.SMEM)] +
                 [pl.BlockSpec(memory_space=pltpu.VMEM)] * 10,
        out_shape=jax.ShapeDtypeStruct((N, 64), jnp.float32),
    )(h, p, eps2.reshape(1), W2a, b2a.reshape(1, -1), g2a.reshape(1, -1),
      bt2a.reshape(1, -1), W2b, b2b.reshape(1, -1), gbn2.reshape(1, -1),
      bbn2.reshape(1, -1), W3, b3.reshape(1, -1))


def kernel(x, edge_index, edge_attr, eps1, We1, be1, W1a, b1a, g1a, bt1a,
           W1b, b1b, gbn1, bbn1, eps2, We2, be2, W2a, b2a, g2a, bt2a,
           W2b, b2b, gbn2, bbn2, W3, b3):
    src = edge_index[0]
    dst = edge_index[1]

    elin1, elin2 = _elin(edge_attr, We1, be1, We2, be2)

    p1 = _edge_kernel_128(x, src, dst, elin1)
    h = _node1(x, p1, eps1, W1a, b1a, g1a, bt1a, W1b, b1b, gbn1, bbn1)

    p2 = _edge_kernel_128(h, src, dst, elin2)
    return _node2(h, p2, eps2, W2a, b2a, g2a, bt2a, W2b, b2b, gbn2, bbn2,
                  W3, b3)


# speedup vs baseline: 4.3811x; 1.5578x over previous
import functools

import jax
import jax.numpy as jnp
from jax import lax
from jax.experimental import pallas as pl
from jax.experimental.pallas import tpu as pltpu
from jax.experimental.pallas import tpu_sc as plsc

N = 10000
E = 320000
D_NODE = 128
D_EDGE = 16

NC = 2
NS = 16
NW = NC * NS
EPW = E // NW
C = 80
G = EPW // C
NP = 10240
RPT = NP // NS
ZR = 40


def _make_edge_kernel(D):
    mesh = plsc.VectorSubcoreMesh(core_axis_name="c", subcore_axis_name="s",
                                  num_cores=NC, num_subcores=NS)

    scratch = [
        pltpu.VMEM((C,), jnp.int32),
        pltpu.VMEM((C,), jnp.int32),
        pltpu.VMEM((C,), jnp.int32),
        pltpu.VMEM((C,), jnp.int32),
        pltpu.VMEM((C, D), jnp.float32),
        pltpu.VMEM((C, D), jnp.float32),
        pltpu.VMEM((C, D), jnp.float32),
        pltpu.VMEM((C, D), jnp.float32),
        pltpu.VMEM((ZR, D), jnp.float32),
        pltpu.VMEM_SHARED((NP, D), jnp.float32),
        pltpu.SemaphoreType.DMA,
        pltpu.SemaphoreType.DMA,
        pltpu.SemaphoreType.DMA,
        pltpu.SemaphoreType.DMA,
        pltpu.SemaphoreType.DMA,
        pltpu.SemaphoreType.DMA,
    ]

    @functools.partial(
        pl.kernel,
        out_type=jax.ShapeDtypeStruct((NC * NP, D), jnp.float32),
        mesh=mesh,
        scratch_types=scratch,
    )
    def edge_kernel(x_hbm, src_hbm, dst_hbm, elin_hbm, out_hbm,
                    src_v0, src_v1, dst_v0, dst_v1, rows_v0, rows_v1,
                    elin_v0, elin_v1, zero_v, aggr_sh,
                    si0, si1, se0, se1, sg0, sg1):
        c = lax.axis_index("c")
        s = lax.axis_index("s")
        nv = D // 16
        src_v = (src_v0, src_v1)
        dst_v = (dst_v0, dst_v1)
        rows_v = (rows_v0, rows_v1)
        elin_v = (elin_v0, elin_v1)
        sem_i = (si0, si1)
        sem_e = (se0, se1)
        sem_g = (sg0, sg1)

        row0 = s * RPT

        def zbody(i, carry):
            for j in range(nv):
                zero_v[i, pl.ds(j * 16, 16)] = jnp.zeros((16,), jnp.float32)
            return carry

        lax.fori_loop(0, ZR, zbody, 0)
        for k in range(RPT // ZR):
            pltpu.sync_copy(zero_v, aggr_sh.at[pl.ds(row0 + k * ZR, ZR)])
        plsc.subcore_barrier()

        wid = c * NS + s
        e0 = wid * EPW
        GL = G - 1

        def fire_idx(b, base):
            pltpu.async_copy(src_hbm.at[pl.ds(base, C)], src_v[b], sem_i[b])
            pltpu.async_copy(dst_hbm.at[pl.ds(base, C)], dst_v[b], sem_i[b])

        def wait_idx(b):
            pltpu.make_async_copy(src_hbm.at[pl.ds(0, C)], src_v[b],
                                  sem_i[b]).wait()
            pltpu.make_async_copy(dst_hbm.at[pl.ds(0, C)], dst_v[b],
                                  sem_i[b]).wait()

        def fire_elin(b, base):
            pltpu.async_copy(elin_hbm.at[pl.ds(base, C)], elin_v[b], sem_e[b])

        def wait_elin(b):
            pltpu.make_async_copy(elin_hbm.at[pl.ds(0, C)], elin_v[b],
                                  sem_e[b]).wait()

        def fire_gather(b):
            pltpu.async_copy(x_hbm.at[src_v[b]], rows_v[b], sem_g[b])

        def wait_gather(b):
            pltpu.make_async_copy(x_hbm.at[src_v[b]], rows_v[b],
                                  sem_g[b]).wait()

        def relu_scatter(b):
            def rbody(r, carry2):
                for j in range(nv):
                    sl = pl.ds(j * 16, 16)
                    rows_v[b][r, sl] = jnp.maximum(
                        rows_v[b][r, sl] + elin_v[b][r, sl], 0.0)
                return carry2

            lax.fori_loop(0, C, rbody, 0)
            pltpu.sync_copy(rows_v[b], aggr_sh.at[dst_v[b]], add=True)

        fire_idx(0, e0)
        fire_idx(1, e0 + C)
        fire_elin(0, e0)
        fire_elin(1, e0 + C)
        wait_idx(0)
        fire_gather(0)

        def step(b, g):
            nb = 1 - b
            wait_gather(b)
            wait_elin(b)
            wait_idx(nb)
            fire_gather(nb)
            relu_scatter(b)
            nxt = e0 + lax.min(g + 2, GL) * C
            fire_idx(b, nxt)
            fire_elin(b, nxt)

        def pbody(k, carry):
            g = 2 * k
            step(0, g)
            step(1, g + 1)
            return carry

        lax.fori_loop(0, G // 2, pbody, 0)

        wait_gather(0)
        wait_elin(0)
        relu_scatter(0)
        wait_idx(1)
        wait_elin(1)

        plsc.subcore_barrier()

        for k in range(RPT // ZR):
            pltpu.sync_copy(aggr_sh.at[pl.ds(row0 + k * ZR, ZR)], zero_v)
            pltpu.sync_copy(zero_v,
                            out_hbm.at[pl.ds(c * NP + row0 + k * ZR, ZR)])

    return edge_kernel


_edge_kernel_128 = _make_edge_kernel(D_NODE)


_BE = 6400


def _elin_body(ea_ref, We1_ref, be1_ref, We2_ref, be2_ref, o1_ref, o2_ref):
    ea = ea_ref[...]
    o1_ref[...] = (jnp.dot(ea, We1_ref[...],
                           preferred_element_type=jnp.float32)
                   + be1_ref[...])
    o2_ref[...] = (jnp.dot(ea, We2_ref[...],
                           preferred_element_type=jnp.float32)
                   + be2_ref[...])


def _elin(ea, We1, be1, We2, be2):
    grid = (E // _BE,)
    return pl.pallas_call(
        _elin_body,
        grid=grid,
        in_specs=[
            pl.BlockSpec((_BE, D_EDGE), lambda i: (i, 0)),
            pl.BlockSpec((D_EDGE, D_NODE), lambda i: (0, 0)),
            pl.BlockSpec((1, D_NODE), lambda i: (0, 0)),
            pl.BlockSpec((D_EDGE, D_NODE), lambda i: (0, 0)),
            pl.BlockSpec((1, D_NODE), lambda i: (0, 0)),
        ],
        out_specs=[
            pl.BlockSpec((_BE, D_NODE), lambda i: (i, 0)),
            pl.BlockSpec((_BE, D_NODE), lambda i: (i, 0)),
        ],
        out_shape=[
            jax.ShapeDtypeStruct((E, D_NODE), jnp.float32),
            jax.ShapeDtypeStruct((E, D_NODE), jnp.float32),
        ],
    )(ea, We1, be1.reshape(1, -1),
      jnp.pad(We2, ((0, 0), (0, D_NODE - 32))),
      jnp.pad(be2, (0, D_NODE - 32)).reshape(1, -1))


def _bn_gelu(t, g, b):
    mu = jnp.mean(t, axis=0, keepdims=True)
    var = jnp.mean(jnp.square(t), axis=0, keepdims=True) - jnp.square(mu)
    t = (t - mu) * lax.rsqrt(var + 1e-5) * g + b
    return 0.5 * t * (1.0 + lax.erf(t * 0.7071067811865476))


def _node1_body(x_ref, p_ref, eps_ref, W1a_ref, b1a_ref, g1a_ref, bt1a_ref,
                W1b_ref, b1b_ref, gbn1_ref, bbn1_ref, o_ref):
    x = x_ref[...]
    z = (1.0 + eps_ref[0]) * x + p_ref[0:N] + p_ref[NP:NP + N]
    t = jnp.dot(z, W1a_ref[...], preferred_element_type=jnp.float32)
    t = t + b1a_ref[...]
    t = _bn_gelu(t, g1a_ref[...], bt1a_ref[...])
    t = jnp.dot(t, W1b_ref[...], preferred_element_type=jnp.float32)
    t = t + b1b_ref[...]
    r = _bn_gelu(t, gbn1_ref[...], bbn1_ref[...])
    r = jnp.concatenate([r, jnp.zeros((N, D_NODE - 32), jnp.float32)], axis=1)
    o_ref[...] = jnp.concatenate(
        [r, jnp.zeros((NP - N, D_NODE), jnp.float32)], axis=0)


def _node1(x, p, eps1, W1a, b1a, g1a, bt1a, W1b, b1b, gbn1, bbn1):
    return pl.pallas_call(
        _node1_body,
        in_specs=[
            pl.BlockSpec(memory_space=pltpu.VMEM),
            pl.BlockSpec(memory_space=pltpu.VMEM),
            pl.BlockSpec(memory_space=pltpu.SMEM),
            pl.BlockSpec(memory_space=pltpu.VMEM),
            pl.BlockSpec(memory_space=pltpu.VMEM),
            pl.BlockSpec(memory_space=pltpu.VMEM),
            pl.BlockSpec(memory_space=pltpu.VMEM),
            pl.BlockSpec(memory_space=pltpu.VMEM),
            pl.BlockSpec(memory_space=pltpu.VMEM),
            pl.BlockSpec(memory_space=pltpu.VMEM),
            pl.BlockSpec(memory_space=pltpu.VMEM),
        ],
        out_shape=jax.ShapeDtypeStruct((NP, D_NODE), jnp.float32),
    )(x, p, eps1.reshape(1), W1a, b1a.reshape(1, -1), g1a.reshape(1, -1),
      bt1a.reshape(1, -1), W1b, b1b.reshape(1, -1), gbn1.reshape(1, -1),
      bbn1.reshape(1, -1))


def _node2_body(h_ref, p_ref, eps_ref, W2a_ref, b2a_ref, g2a_ref, bt2a_ref,
                W2b_ref, b2b_ref, gbn2_ref, bbn2_ref, W3_ref, b3_ref, o_ref):
    h = h_ref[0:N, 0:32]
    z = (1.0 + eps_ref[0]) * h + p_ref[0:N, 0:32] + p_ref[NP:NP + N, 0:32]
    t = jnp.dot(z, W2a_ref[...], preferred_element_type=jnp.float32)
    t = t + b2a_ref[...]
    t = _bn_gelu(t, g2a_ref[...], bt2a_ref[...])
    t = jnp.dot(t, W2b_ref[...], preferred_element_type=jnp.float32)
    t = t + b2b_ref[...]
    t = _bn_gelu(t, gbn2_ref[...], bbn2_ref[...])
    o_ref[...] = (jnp.dot(t, W3_ref[...],
                          preferred_element_type=jnp.float32)
                  + b3_ref[...])


def _node2(h, p, eps2, W2a, b2a, g2a, bt2a, W2b, b2b, gbn2, bbn2, W3, b3):
    return pl.pallas_call(
        _node2_body,
        in_specs=[pl.BlockSpec(memory_space=pltpu.VMEM),
                  pl.BlockSpec(memory_space=pltpu.VMEM),
                  pl.BlockSpec(memory_space=pltpu.SMEM)] +
                 [pl.BlockSpec(memory_space=pltpu.VMEM)] * 10,
        out_shape=jax.ShapeDtypeStruct((N, 64), jnp.float32),
    )(h, p, eps2.reshape(1), W2a, b2a.reshape(1, -1), g2a.reshape(1, -1),
      bt2a.reshape(1, -1), W2b, b2b.reshape(1, -1), gbn2.reshape(1, -1),
      bbn2.reshape(1, -1), W3, b3.reshape(1, -1))


def kernel(x, edge_index, edge_attr, eps1, We1, be1, W1a, b1a, g1a, bt1a,
           W1b, b1b, gbn1, bbn1, eps2, We2, be2, W2a, b2a, g2a, bt2a,
           W2b, b2b, gbn2, bbn2, W3, b3):
    src = edge_index[0]
    dst = edge_index[1]

    elin1, elin2 = _elin(edge_attr, We1, be1, We2, be2)

    p1 = _edge_kernel_128(x, src, dst, elin1)
    h = _node1(x, p1, eps1, W1a, b1a, g1a, bt1a, W1b, b1b, gbn1, bbn1)

    p2 = _edge_kernel_128(h, src, dst, elin2)
    return _node2(h, p2, eps2, W2a, b2a, g2a, bt2a, W2b, b2b, gbn2, bbn2,
                  W3, b3)
